# baseline (speedup 1.0000x reference)
import functools

import jax
import jax.numpy as jnp
from jax import lax
from jax.experimental import pallas as pl
from jax.experimental.pallas import tpu as pltpu
from jax.experimental.pallas import tpu_sc as plsc

_C = 8
_R = 128
_V = _R * _R * _R
_NC = 2
_NS = 16
_NW = _NC * _NS
_L = 16
_B = 128


def _interp_body(xs, ys, zs, table, out, cv, fv, idxv, rowsv, outv,
                 csem0, csem1, gsem0, gsem1, osem0, osem1):
  n_points = xs.shape[0]
  pts_per_w = n_points // _NW
  n_chunks = pts_per_w // _B
  n_pairs = n_chunks // 2
  wid = lax.axis_index("s") * _NC + lax.axis_index("c")
  w_base = wid * pts_per_w

  csems = (csem0, csem1)
  gsems = (gsem0, gsem1)
  osems = (osem0, osem1)

  def coord_copies(t, par):
    base = w_base + t * _B
    sem = csems[par]
    return [
        pltpu.make_async_copy(xs.at[pl.ds(base, _B)], cv.at[par, 0], sem),
        pltpu.make_async_copy(ys.at[pl.ds(base, _B)], cv.at[par, 1], sem),
        pltpu.make_async_copy(zs.at[pl.ds(base, _B)], cv.at[par, 2], sem),
    ]

  def gather_copies(par):
    sem = gsems[par]
    return [
        pltpu.make_async_copy(table.at[idxv.at[par, k]], rowsv.at[par, k], sem)
        for k in range(4)
    ]

  def out_copy(t, par):
    base = w_base + t * _B
    return pltpu.make_async_copy(
        outv.at[par], out.at[pl.ds(base * _C, _B * _C)], osems[par])

  def fire(copies):
    for c in copies:
      c.start()

  def drain(copies):
    for c in copies:
      c.wait()

  def index_phase(par):
    def index_body(j, _):
      sl = pl.ds(j * _L, _L)
      sx = cv[par, 0, sl]
      sy = cv[par, 1, sl]
      sz = cv[par, 2, sl]
      ix = jnp.minimum(sx.astype(jnp.int32), _R - 2)
      iy = jnp.minimum(sy.astype(jnp.int32), _R - 2)
      iz = jnp.minimum(sz.astype(jnp.int32), _R - 2)
      fv[par, 0, sl] = sx - ix.astype(jnp.float32)
      fv[par, 1, sl] = sy - iy.astype(jnp.float32)
      fv[par, 2, sl] = sz - iz.astype(jnp.float32)
      cell = (iz << 14) + (iy << 7) + ix
      prow = (cell >> 1) + ((cell & 1) << 20)
      idxv[par, 0, sl] = prow
      idxv[par, 1, sl] = prow + 64
      idxv[par, 2, sl] = prow + 8192
      idxv[par, 3, sl] = prow + 8256
      return 0

    lax.fori_loop(0, _B // _L, index_body, 0)

  lane = lax.iota(jnp.int32, _L)
  lo_half = lane < _C
  swap = lane ^ _C
  hi_f = (lane >> 3).astype(jnp.float32)

  def combine_phase(par):
    def combine_body(jj, _):
      sl = pl.ds(jj * _L, _L)
      fxr = fv[par, 0, sl]
      fyr = fv[par, 1, sl]
      fzr = fv[par, 2, sl]
      res = []
      for u in range(_L):
        splat = lane * 0 + u
        ex = fxr[splat]
        ey = fyr[splat]
        ez = fzr[splat]
        wx = jnp.where(lo_half, 1.0 - ex, ex)
        eyc = 1.0 - ey
        ezc = 1.0 - ez
        w00 = eyc * ezc
        w01 = ey * ezc
        w10 = eyc * ez
        w11 = ey * ez
        p = jj * _L + u
        tsum = (w00 * rowsv[par, 0, p, :] + w01 * rowsv[par, 1, p, :]
                + w10 * rowsv[par, 2, p, :] + w11 * rowsv[par, 3, p, :])
        acc = wx * tsum
        res.append(acc + acc[swap])
      m = [jnp.where(lo_half, res[q], res[q + 8]) for q in range(_C)]
      for b in range(3):
        nxt = list(m)
        rot = lane ^ (1 << b)
        bit = (lane >> b) & 1
        for u in range(_C):
          if u & (1 << b):
            continue
          v = u | (1 << b)
          xu, xv = m[u], m[v]
          nxt[u] = jnp.where(bit == 0, xu, xv[rot])
          nxt[v] = jnp.where(bit == 0, xu[rot], xv)
        m = nxt
      for c in range(_C):
        outv[par, pl.ds(c * _B + jj * _L, _L)] = m[c]
      return 0

    lax.fori_loop(0, _B // _L, combine_body, 0)

  fire(coord_copies(0, 0))
  fire(coord_copies(1, 1))

  def pair_body(m, _):
    a = 2 * m
    b = a + 1

    drain(coord_copies(a, 0))
    index_phase(0)
    fire(gather_copies(0))

    @pl.when(m + 1 < n_pairs)
    def _():
      fire(coord_copies(a + 2, 0))

    @pl.when(m > 0)
    def _():
      drain(gather_copies(1))

      @pl.when(m > 1)
      def _():
        drain([out_copy(b - 4, 1)])

      combine_phase(1)
      fire([out_copy(b - 2, 1)])

    drain(coord_copies(b, 1))
    index_phase(1)
    fire(gather_copies(1))

    @pl.when(m + 1 < n_pairs)
    def _():
      fire(coord_copies(b + 2, 1))

    drain(gather_copies(0))

    @pl.when(m > 0)
    def _():
      drain([out_copy(a - 2, 0)])

    combine_phase(0)
    fire([out_copy(a, 0)])
    return 0

  lax.fori_loop(0, n_pairs, pair_body, 0)

  last = n_chunks - 1
  drain(gather_copies(1))
  drain([out_copy(last - 2, 1)])
  combine_phase(1)
  fire([out_copy(last, 1)])
  drain([out_copy(last - 1, 0)])
  drain([out_copy(last, 1)])


_TB = 2048


def _table_body(g8v, table, inv, outve, outvo, isem0, isem1, osem0, osem1):
  nv = g8v.shape[1]
  cells_pw = nv // _NW
  n_ch = cells_pw // _TB
  nrow = _TB // 2
  wid = lax.axis_index("s") * _NC + lax.axis_index("c")
  cw = wid * cells_pw
  isems = (isem0, isem1)
  osems = (osem0, osem1)
  lane = lax.iota(jnp.int32, _L)
  lo_half = lane < _C
  swap = lane ^ _C

  def in_copies(t, par):
    c0 = cw + t * _TB
    ce = jnp.minimum(c0 + _TB, nv - _L)
    cps = [
        pltpu.make_async_copy(
            g8v.at[c, pl.ds(c0, _TB)], inv.at[par, c, pl.ds(0, _TB)],
            isems[par])
        for c in range(_C)
    ]
    cps += [
        pltpu.make_async_copy(
            g8v.at[c, pl.ds(ce, _L)], inv.at[par, c, pl.ds(_TB, _L)],
            isems[par])
        for c in range(_C)
    ]
    return cps

  def out_copies(t, par):
    r0 = (cw + t * _TB) // 2
    return [
        pltpu.make_async_copy(
            outve.at[par, pl.ds(0, nrow)], table.at[pl.ds(r0, nrow), :],
            osems[par]),
        pltpu.make_async_copy(
            outvo.at[par], table.at[pl.ds(nv // 2 + r0, nrow), :],
            osems[par]),
    ]

  def compute(par):
    def body(j, _):
      sl = pl.ds(j * _L, _L)
      m = [inv[par, c, sl] for c in range(_C)]
      for b in range(3):
        nxt = list(m)
        rot = lane ^ (1 << b)
        bit = (lane >> b) & 1
        for u in range(_C):
          if u & (1 << b):
            continue
          v = u | (1 << b)
          xu, xv = m[u], m[v]
          nxt[u] = jnp.where(bit == 0, xu, xv[rot])
          nxt[v] = jnp.where(bit == 0, xu[rot], xv)
        m = nxt
      nxt = list(m)
      bit = lane >> 3
      for u in range(0, _C, 2):
        xu, xv = m[u], m[u + 1]
        nxt[u] = jnp.where(bit == 0, xu, xv[swap])
        nxt[u + 1] = jnp.where(bit == 0, xu[swap], xv)
      m = nxt
      for q in range(_C):
        jrow = ((q & 1) << 2) | ((q >> 2) << 1) | ((q >> 1) & 1)
        outve[par, 8 * j + jrow, :] = m[q]
      return 0

    lax.fori_loop(0, _TB // _L + 1, body, 0)

    def odd_body(r, _):
      e0 = outve[par, r, :]
      e1 = outve[par, r + 1, :]
      outvo[par, r, :] = jnp.where(lo_half, e0[swap], e1[swap])
      return 0

    lax.fori_loop(0, nrow, odd_body, 0)

  fire = lambda cps: [c.start() for c in cps]
  drain = lambda cps: [c.wait() for c in cps]
  fire(in_copies(0, 0))
  fire(in_copies(1, 1))

  def pair_body(mm, _):
    a = 2 * mm
    b = a + 1
    drain(in_copies(a, 0))

    @pl.when(mm > 0)
    def _():
      drain(out_copies(a - 2, 0))

    compute(0)

    @pl.when(mm + 1 < n_ch // 2)
    def _():
      fire(in_copies(a + 2, 0))

    fire(out_copies(a, 0))
    drain(in_copies(b, 1))

    @pl.when(mm > 0)
    def _():
      drain(out_copies(b - 2, 1))

    compute(1)

    @pl.when(mm + 1 < n_ch // 2)
    def _():
      fire(in_copies(b + 2, 1))

    fire(out_copies(b, 1))
    return 0

  lax.fori_loop(0, n_ch // 2, pair_body, 0)
  drain(out_copies(n_ch - 2, 0))
  drain(out_copies(n_ch - 1, 1))


@functools.lru_cache(maxsize=None)
def _build_table():
  mesh = plsc.VectorSubcoreMesh(
      core_axis_name="c", subcore_axis_name="s",
      num_cores=_NC, num_subcores=_NS)
  return pl.kernel(
      _table_body,
      out_type=jax.ShapeDtypeStruct((_V, 2 * _C), jnp.float32),
      mesh=mesh,
      compiler_params=pltpu.CompilerParams(use_tc_tiling_on_sc=False),
      scratch_types=[
          pltpu.VMEM((2, _C, _TB + _L), jnp.float32),
          pltpu.VMEM((2, _TB // 2 + 8, 2 * _C), jnp.float32),
          pltpu.VMEM((2, _TB // 2, 2 * _C), jnp.float32),
          pltpu.SemaphoreType.DMA,
          pltpu.SemaphoreType.DMA,
          pltpu.SemaphoreType.DMA,
          pltpu.SemaphoreType.DMA,
      ],
  )


@functools.lru_cache(maxsize=None)
def _build(n_points):
  assert n_points % (_NW * _B * 2) == 0
  mesh = plsc.VectorSubcoreMesh(
      core_axis_name="c", subcore_axis_name="s",
      num_cores=_NC, num_subcores=_NS)
  return pl.kernel(
      _interp_body,
      out_type=jax.ShapeDtypeStruct((n_points * _C,), jnp.float32),
      mesh=mesh,
      compiler_params=pltpu.CompilerParams(use_tc_tiling_on_sc=False),
      scratch_types=[
          pltpu.VMEM((2, 3, _B), jnp.float32),
          pltpu.VMEM((2, 3, _B), jnp.float32),
          pltpu.VMEM((2, 4, _B), jnp.int32),
          pltpu.VMEM((2, 4, _B, 2 * _C), jnp.float32),
          pltpu.VMEM((2, _B * _C), jnp.float32),
          pltpu.SemaphoreType.DMA,
          pltpu.SemaphoreType.DMA,
          pltpu.SemaphoreType.DMA,
          pltpu.SemaphoreType.DMA,
          pltpu.SemaphoreType.DMA,
          pltpu.SemaphoreType.DMA,
      ],
  )


def kernel(x, grid, bound):
  n = x.shape[0]
  half = 0.5 * (_R - 1)
  xsc = (x.astype(jnp.float32) * (half / bound) + half).T
  table = _build_table()(grid.reshape(_C, _V))
  out = _build(n)(xsc[0], xsc[1], xsc[2], table)
  return out.reshape(n // _B, _C, _B).swapaxes(1, 2).reshape(n, _C)


# speedup vs baseline: 1.0053x; 1.0053x over previous
import functools

import jax
import jax.numpy as jnp
from jax import lax
from jax.experimental import pallas as pl
from jax.experimental.pallas import tpu as pltpu
from jax.experimental.pallas import tpu_sc as plsc

_C = 8
_R = 128
_V = _R * _R * _R
_NC = 2
_NS = 16
_NW = _NC * _NS
_L = 16
_B = 256
_NH = _B // 128


def _interp_body(xs, ys, zs, table, out, cv, fv, idxv, rowsv, outv,
                 csem0, csem1, gsem0, gsem1, osem0, osem1):
  n_points = xs.shape[0]
  pts_per_w = n_points // _NW
  n_chunks = pts_per_w // _B
  n_pairs = n_chunks // 2
  wid = lax.axis_index("s") * _NC + lax.axis_index("c")
  w_base = wid * pts_per_w

  csems = (csem0, csem1)
  gsems = (gsem0, gsem1)
  osems = (osem0, osem1)

  def coord_copies(t, par):
    base = w_base + t * _B
    sem = csems[par]
    return [
        pltpu.make_async_copy(xs.at[pl.ds(base, _B)], cv.at[par, 0], sem),
        pltpu.make_async_copy(ys.at[pl.ds(base, _B)], cv.at[par, 1], sem),
        pltpu.make_async_copy(zs.at[pl.ds(base, _B)], cv.at[par, 2], sem),
    ]

  def gather_copies(par):
    sem = gsems[par]
    return [
        pltpu.make_async_copy(
            table.at[idxv.at[par, k, h]],
            rowsv.at[par, k, pl.ds(h * 128, 128)], sem)
        for k in range(4) for h in range(_NH)
    ]

  def out_copy(t, par):
    base = w_base + t * _B
    return pltpu.make_async_copy(
        outv.at[par], out.at[pl.ds(base * _C, _B * _C)], osems[par])

  def fire(copies):
    for c in copies:
      c.start()

  def drain(copies):
    for c in copies:
      c.wait()

  def index_phase(par):
    def index_body(j, _):
      sl = pl.ds(j * _L, _L)
      sx = cv[par, 0, sl]
      sy = cv[par, 1, sl]
      sz = cv[par, 2, sl]
      ix = jnp.minimum(sx.astype(jnp.int32), _R - 2)
      iy = jnp.minimum(sy.astype(jnp.int32), _R - 2)
      iz = jnp.minimum(sz.astype(jnp.int32), _R - 2)
      fv[par, 0, sl] = sx - ix.astype(jnp.float32)
      fv[par, 1, sl] = sy - iy.astype(jnp.float32)
      fv[par, 2, sl] = sz - iz.astype(jnp.float32)
      cell = (iz << 14) + (iy << 7) + ix
      prow = (cell >> 1) + ((cell & 1) << 20)
      jh = j >> 3
      js = pl.ds((j & 7) * _L, _L)
      idxv[par, 0, jh, js] = prow
      idxv[par, 1, jh, js] = prow + 64
      idxv[par, 2, jh, js] = prow + 8192
      idxv[par, 3, jh, js] = prow + 8256
      return 0

    lax.fori_loop(0, _B // _L, index_body, 0)

  lane = lax.iota(jnp.int32, _L)
  lo_half = lane < _C
  swap = lane ^ _C
  hi_f = (lane >> 3).astype(jnp.float32)

  def combine_phase(par):
    def combine_body(jj, _):
      sl = pl.ds(jj * _L, _L)
      fxr = fv[par, 0, sl]
      fyr = fv[par, 1, sl]
      fzr = fv[par, 2, sl]
      res = []
      for u in range(_L):
        splat = lane * 0 + u
        ex = fxr[splat]
        ey = fyr[splat]
        ez = fzr[splat]
        wx = jnp.where(lo_half, 1.0 - ex, ex)
        eyc = 1.0 - ey
        ezc = 1.0 - ez
        w00 = eyc * ezc
        w01 = ey * ezc
        w10 = eyc * ez
        w11 = ey * ez
        p = jj * _L + u
        tsum = (w00 * rowsv[par, 0, p, :] + w01 * rowsv[par, 1, p, :]
                + w10 * rowsv[par, 2, p, :] + w11 * rowsv[par, 3, p, :])
        acc = wx * tsum
        res.append(acc + acc[swap])
      m = [jnp.where(lo_half, res[q], res[q + 8]) for q in range(_C)]
      for b in range(3):
        nxt = list(m)
        rot = lane ^ (1 << b)
        bit = (lane >> b) & 1
        for u in range(_C):
          if u & (1 << b):
            continue
          v = u | (1 << b)
          xu, xv = m[u], m[v]
          nxt[u] = jnp.where(bit == 0, xu, xv[rot])
          nxt[v] = jnp.where(bit == 0, xu[rot], xv)
        m = nxt
      off = ((jj >> 3) << 10) + ((jj & 7) << 4)
      for c in range(_C):
        outv[par, pl.ds(off + c * 128, _L)] = m[c]
      return 0

    lax.fori_loop(0, _B // _L, combine_body, 0)

  fire(coord_copies(0, 0))
  fire(coord_copies(1, 1))

  def pair_body(m, _):
    a = 2 * m
    b = a + 1

    drain(coord_copies(a, 0))
    index_phase(0)
    fire(gather_copies(0))

    @pl.when(m + 1 < n_pairs)
    def _():
      fire(coord_copies(a + 2, 0))

    @pl.when(m > 0)
    def _():
      drain(gather_copies(1))

      @pl.when(m > 1)
      def _():
        drain([out_copy(b - 4, 1)])

      combine_phase(1)
      fire([out_copy(b - 2, 1)])

    drain(coord_copies(b, 1))
    index_phase(1)
    fire(gather_copies(1))

    @pl.when(m + 1 < n_pairs)
    def _():
      fire(coord_copies(b + 2, 1))

    drain(gather_copies(0))

    @pl.when(m > 0)
    def _():
      drain([out_copy(a - 2, 0)])

    combine_phase(0)
    fire([out_copy(a, 0)])
    return 0

  lax.fori_loop(0, n_pairs, pair_body, 0)

  last = n_chunks - 1
  drain(gather_copies(1))
  drain([out_copy(last - 2, 1)])
  combine_phase(1)
  fire([out_copy(last, 1)])
  drain([out_copy(last - 1, 0)])
  drain([out_copy(last, 1)])


_TB = 2048


def _table_body(g8v, table, inv, outve, outvo, isem0, isem1, osem0, osem1):
  nv = g8v.shape[1]
  cells_pw = nv // _NW
  n_ch = cells_pw // _TB
  nrow = _TB // 2
  wid = lax.axis_index("s") * _NC + lax.axis_index("c")
  cw = wid * cells_pw
  isems = (isem0, isem1)
  osems = (osem0, osem1)
  lane = lax.iota(jnp.int32, _L)
  lo_half = lane < _C
  swap = lane ^ _C

  def in_copies(t, par):
    c0 = cw + t * _TB
    ce = jnp.minimum(c0 + _TB, nv - _L)
    cps = [
        pltpu.make_async_copy(
            g8v.at[c, pl.ds(c0, _TB)], inv.at[par, c, pl.ds(0, _TB)],
            isems[par])
        for c in range(_C)
    ]
    cps += [
        pltpu.make_async_copy(
            g8v.at[c, pl.ds(ce, _L)], inv.at[par, c, pl.ds(_TB, _L)],
            isems[par])
        for c in range(_C)
    ]
    return cps

  def out_copies(t, par):
    r0 = (cw + t * _TB) // 2
    return [
        pltpu.make_async_copy(
            outve.at[par, pl.ds(0, nrow)], table.at[pl.ds(r0, nrow), :],
            osems[par]),
        pltpu.make_async_copy(
            outvo.at[par], table.at[pl.ds(nv // 2 + r0, nrow), :],
            osems[par]),
    ]

  def compute(par):
    def body(j, _):
      sl = pl.ds(j * _L, _L)
      m = [inv[par, c, sl] for c in range(_C)]
      for b in range(3):
        nxt = list(m)
        rot = lane ^ (1 << b)
        bit = (lane >> b) & 1
        for u in range(_C):
          if u & (1 << b):
            continue
          v = u | (1 << b)
          xu, xv = m[u], m[v]
          nxt[u] = jnp.where(bit == 0, xu, xv[rot])
          nxt[v] = jnp.where(bit == 0, xu[rot], xv)
        m = nxt
      nxt = list(m)
      bit = lane >> 3
      for u in range(0, _C, 2):
        xu, xv = m[u], m[u + 1]
        nxt[u] = jnp.where(bit == 0, xu, xv[swap])
        nxt[u + 1] = jnp.where(bit == 0, xu[swap], xv)
      m = nxt
      for q in range(_C):
        jrow = ((q & 1) << 2) | ((q >> 2) << 1) | ((q >> 1) & 1)
        outve[par, 8 * j + jrow, :] = m[q]
      return 0

    lax.fori_loop(0, _TB // _L + 1, body, 0)

    def odd_body(r, _):
      e0 = outve[par, r, :]
      e1 = outve[par, r + 1, :]
      outvo[par, r, :] = jnp.where(lo_half, e0[swap], e1[swap])
      return 0

    lax.fori_loop(0, nrow, odd_body, 0)

  fire = lambda cps: [c.start() for c in cps]
  drain = lambda cps: [c.wait() for c in cps]
  fire(in_copies(0, 0))
  fire(in_copies(1, 1))

  def pair_body(mm, _):
    a = 2 * mm
    b = a + 1
    drain(in_copies(a, 0))

    @pl.when(mm > 0)
    def _():
      drain(out_copies(a - 2, 0))

    compute(0)

    @pl.when(mm + 1 < n_ch // 2)
    def _():
      fire(in_copies(a + 2, 0))

    fire(out_copies(a, 0))
    drain(in_copies(b, 1))

    @pl.when(mm > 0)
    def _():
      drain(out_copies(b - 2, 1))

    compute(1)

    @pl.when(mm + 1 < n_ch // 2)
    def _():
      fire(in_copies(b + 2, 1))

    fire(out_copies(b, 1))
    return 0

  lax.fori_loop(0, n_ch // 2, pair_body, 0)
  drain(out_copies(n_ch - 2, 0))
  drain(out_copies(n_ch - 1, 1))


@functools.lru_cache(maxsize=None)
def _build_table():
  mesh = plsc.VectorSubcoreMesh(
      core_axis_name="c", subcore_axis_name="s",
      num_cores=_NC, num_subcores=_NS)
  return pl.kernel(
      _table_body,
      out_type=jax.ShapeDtypeStruct((_V, 2 * _C), jnp.float32),
      mesh=mesh,
      compiler_params=pltpu.CompilerParams(use_tc_tiling_on_sc=False),
      scratch_types=[
          pltpu.VMEM((2, _C, _TB + _L), jnp.float32),
          pltpu.VMEM((2, _TB // 2 + 8, 2 * _C), jnp.float32),
          pltpu.VMEM((2, _TB // 2, 2 * _C), jnp.float32),
          pltpu.SemaphoreType.DMA,
          pltpu.SemaphoreType.DMA,
          pltpu.SemaphoreType.DMA,
          pltpu.SemaphoreType.DMA,
      ],
  )


@functools.lru_cache(maxsize=None)
def _build(n_points):
  assert n_points % (_NW * _B * 2) == 0
  mesh = plsc.VectorSubcoreMesh(
      core_axis_name="c", subcore_axis_name="s",
      num_cores=_NC, num_subcores=_NS)
  return pl.kernel(
      _interp_body,
      out_type=jax.ShapeDtypeStruct((n_points * _C,), jnp.float32),
      mesh=mesh,
      compiler_params=pltpu.CompilerParams(use_tc_tiling_on_sc=False),
      scratch_types=[
          pltpu.VMEM((2, 3, _B), jnp.float32),
          pltpu.VMEM((2, 3, _B), jnp.float32),
          pltpu.VMEM((2, 4, _NH, 128), jnp.int32),
          pltpu.VMEM((2, 4, _B, 2 * _C), jnp.float32),
          pltpu.VMEM((2, _B * _C), jnp.float32),
          pltpu.SemaphoreType.DMA,
          pltpu.SemaphoreType.DMA,
          pltpu.SemaphoreType.DMA,
          pltpu.SemaphoreType.DMA,
          pltpu.SemaphoreType.DMA,
          pltpu.SemaphoreType.DMA,
      ],
  )


def kernel(x, grid, bound):
  n = x.shape[0]
  half = 0.5 * (_R - 1)
  xsc = (x.astype(jnp.float32) * (half / bound) + half).T
  table = _build_table()(grid.reshape(_C, _V))
  out = _build(n)(xsc[0], xsc[1], xsc[2], table)
  return out.reshape(n // _B, _C, _B).swapaxes(1, 2).reshape(n, _C)


# speedup vs baseline: 1.2251x; 1.2187x over previous
import functools

import jax
import jax.numpy as jnp
from jax import lax
from jax.experimental import pallas as pl
from jax.experimental.pallas import tpu as pltpu
from jax.experimental.pallas import tpu_sc as plsc

_C = 8
_R = 128
_V = _R * _R * _R
_NC = 2
_NS = 16
_NW = _NC * _NS
_L = 16
_B = 128


def _interp_body(xs, ys, zs, table, out, cv, fv, idxv, rowsv, outv,
                 csem0, csem1, gsem0, gsem1, osem0, osem1):
  n_points = xs.shape[0]
  pts_per_w = n_points // _NW
  n_chunks = pts_per_w // _B
  n_pairs = n_chunks // 2
  wid = lax.axis_index("s") * _NC + lax.axis_index("c")
  w_base = wid * pts_per_w

  csems = (csem0, csem1)
  gsems = (gsem0, gsem1)
  osems = (osem0, osem1)

  def coord_copies(t, par):
    base = w_base + t * _B
    sem = csems[par]
    return [
        pltpu.make_async_copy(xs.at[pl.ds(base, _B)], cv.at[par, 0], sem),
        pltpu.make_async_copy(ys.at[pl.ds(base, _B)], cv.at[par, 1], sem),
        pltpu.make_async_copy(zs.at[pl.ds(base, _B)], cv.at[par, 2], sem),
    ]

  def gather_copies(par):
    sem = gsems[par]
    return [
        pltpu.make_async_copy(table.at[idxv.at[par, k]], rowsv.at[par, k], sem)
        for k in range(4)
    ]

  def out_copy(t, par):
    base = w_base + t * _B
    return pltpu.make_async_copy(
        outv.at[par], out.at[pl.ds(base * _C, _B * _C)], osems[par])

  def fire(copies):
    for c in copies:
      c.start()

  def drain(copies):
    for c in copies:
      c.wait()

  def index_phase(par):
    def index_body(j, _):
      sl = pl.ds(j * _L, _L)
      sx = cv[par, 0, sl]
      sy = cv[par, 1, sl]
      sz = cv[par, 2, sl]
      ix = jnp.minimum(sx.astype(jnp.int32), _R - 2)
      iy = jnp.minimum(sy.astype(jnp.int32), _R - 2)
      iz = jnp.minimum(sz.astype(jnp.int32), _R - 2)
      fv[par, 0, sl] = sx - ix.astype(jnp.float32)
      fv[par, 1, sl] = sy - iy.astype(jnp.float32)
      fv[par, 2, sl] = sz - iz.astype(jnp.float32)
      cell = (iz << 14) + (iy << 7) + ix
      prow = (cell >> 1) + ((cell & 1) << 20)
      idxv[par, 0, sl] = prow
      idxv[par, 1, sl] = prow + 64
      idxv[par, 2, sl] = prow + 8192
      idxv[par, 3, sl] = prow + 8256
      return 0

    lax.fori_loop(0, _B // _L, index_body, 0)

  lane = lax.iota(jnp.int32, _L)
  lo_half = lane < _C
  swap = lane ^ _C
  hi_f = (lane >> 3).astype(jnp.float32)

  def combine_phase(par):
    def combine_body(jj, _):
      sl = pl.ds(jj * _L, _L)
      fxr = fv[par, 0, sl]
      fyr = fv[par, 1, sl]
      fzr = fv[par, 2, sl]
      res = []
      for u in range(_L):
        splat = lane * 0 + u
        ex = fxr[splat]
        ey = fyr[splat]
        ez = fzr[splat]
        wx = jnp.where(lo_half, 1.0 - ex, ex)
        eyc = 1.0 - ey
        ezc = 1.0 - ez
        w00 = eyc * ezc
        w01 = ey * ezc
        w10 = eyc * ez
        w11 = ey * ez
        p = jj * _L + u
        tsum = (w00 * rowsv[par, 0, p, :] + w01 * rowsv[par, 1, p, :]
                + w10 * rowsv[par, 2, p, :] + w11 * rowsv[par, 3, p, :])
        acc = wx * tsum
        res.append(acc + acc[swap])
      m = [jnp.where(lo_half, res[q], res[q + 8]) for q in range(_C)]
      for b in range(3):
        nxt = list(m)
        rot = lane ^ (1 << b)
        bit = (lane >> b) & 1
        for u in range(_C):
          if u & (1 << b):
            continue
          v = u | (1 << b)
          xu, xv = m[u], m[v]
          nxt[u] = jnp.where(bit == 0, xu, xv[rot])
          nxt[v] = jnp.where(bit == 0, xu[rot], xv)
        m = nxt
      for c in range(_C):
        outv[par, pl.ds(c * _B + jj * _L, _L)] = m[c]
      return 0

    lax.fori_loop(0, _B // _L, combine_body, 0)

  fire(coord_copies(0, 0))
  fire(coord_copies(1, 1))

  def pair_body(m, _):
    a = 2 * m
    b = a + 1

    drain(coord_copies(a, 0))
    index_phase(0)
    fire(gather_copies(0))

    @pl.when(m + 1 < n_pairs)
    def _():
      fire(coord_copies(a + 2, 0))

    @pl.when(m > 0)
    def _():
      drain(gather_copies(1))

      @pl.when(m > 1)
      def _():
        drain([out_copy(b - 4, 1)])

      combine_phase(1)
      fire([out_copy(b - 2, 1)])

    drain(coord_copies(b, 1))
    index_phase(1)
    fire(gather_copies(1))

    @pl.when(m + 1 < n_pairs)
    def _():
      fire(coord_copies(b + 2, 1))

    drain(gather_copies(0))

    @pl.when(m > 0)
    def _():
      drain([out_copy(a - 2, 0)])

    combine_phase(0)
    fire([out_copy(a, 0)])
    return 0

  lax.fori_loop(0, n_pairs, pair_body, 0)

  last = n_chunks - 1
  drain(gather_copies(1))
  drain([out_copy(last - 2, 1)])
  combine_phase(1)
  fire([out_copy(last, 1)])
  drain([out_copy(last - 1, 0)])
  drain([out_copy(last, 1)])


_TB = 2048


def _table_body(g8v, table, inv, outve, outvo, isem0, isem1, osem0, osem1):
  nv = g8v.shape[1]
  cells_pw = nv // _NW
  n_ch = cells_pw // _TB
  nrow = _TB // 2
  wid = lax.axis_index("s") * _NC + lax.axis_index("c")
  cw = wid * cells_pw
  isems = (isem0, isem1)
  osems = (osem0, osem1)
  lane = lax.iota(jnp.int32, _L)
  lo_half = lane < _C
  swap = lane ^ _C

  def in_copies(t, par):
    c0 = cw + t * _TB
    ce = jnp.minimum(c0 + _TB, nv - _L)
    cps = [
        pltpu.make_async_copy(
            g8v.at[c, pl.ds(c0, _TB)], inv.at[par, c, pl.ds(0, _TB)],
            isems[par])
        for c in range(_C)
    ]
    cps += [
        pltpu.make_async_copy(
            g8v.at[c, pl.ds(ce, _L)], inv.at[par, c, pl.ds(_TB, _L)],
            isems[par])
        for c in range(_C)
    ]
    return cps

  def out_copies(t, par):
    r0 = (cw + t * _TB) // 2
    return [
        pltpu.make_async_copy(
            outve.at[par, pl.ds(0, nrow)], table.at[pl.ds(r0, nrow), :],
            osems[par]),
        pltpu.make_async_copy(
            outvo.at[par], table.at[pl.ds(nv // 2 + r0, nrow), :],
            osems[par]),
    ]

  def compute(par):
    def body(j, _):
      sl = pl.ds(j * _L, _L)
      m = [inv[par, c, sl] for c in range(_C)]
      for b in range(3):
        nxt = list(m)
        rot = lane ^ (1 << b)
        bit = (lane >> b) & 1
        for u in range(_C):
          if u & (1 << b):
            continue
          v = u | (1 << b)
          xu, xv = m[u], m[v]
          nxt[u] = jnp.where(bit == 0, xu, xv[rot])
          nxt[v] = jnp.where(bit == 0, xu[rot], xv)
        m = nxt
      nxt = list(m)
      bit = lane >> 3
      for u in range(0, _C, 2):
        xu, xv = m[u], m[u + 1]
        nxt[u] = jnp.where(bit == 0, xu, xv[swap])
        nxt[u + 1] = jnp.where(bit == 0, xu[swap], xv)
      m = nxt
      for q in range(_C):
        jrow = ((q & 1) << 2) | ((q >> 2) << 1) | ((q >> 1) & 1)
        outve[par, 8 * j + jrow, :] = m[q]
      return 0

    lax.fori_loop(0, _TB // _L + 1, body, 0)

    def odd_body(r, _):
      base = r * 8
      sw = [outve[par, base + i, :][swap] for i in range(9)]
      for i in range(8):
        outvo[par, base + i, :] = jnp.where(lo_half, sw[i], sw[i + 1])
      return 0

    lax.fori_loop(0, nrow // 8, odd_body, 0)

  fire = lambda cps: [c.start() for c in cps]
  drain = lambda cps: [c.wait() for c in cps]
  fire(in_copies(0, 0))
  fire(in_copies(1, 1))

  def pair_body(mm, _):
    a = 2 * mm
    b = a + 1
    drain(in_copies(a, 0))

    @pl.when(mm > 0)
    def _():
      drain(out_copies(a - 2, 0))

    compute(0)

    @pl.when(mm + 1 < n_ch // 2)
    def _():
      fire(in_copies(a + 2, 0))

    fire(out_copies(a, 0))
    drain(in_copies(b, 1))

    @pl.when(mm > 0)
    def _():
      drain(out_copies(b - 2, 1))

    compute(1)

    @pl.when(mm + 1 < n_ch // 2)
    def _():
      fire(in_copies(b + 2, 1))

    fire(out_copies(b, 1))
    return 0

  lax.fori_loop(0, n_ch // 2, pair_body, 0)
  drain(out_copies(n_ch - 2, 0))
  drain(out_copies(n_ch - 1, 1))


@functools.lru_cache(maxsize=None)
def _build_table():
  mesh = plsc.VectorSubcoreMesh(
      core_axis_name="c", subcore_axis_name="s",
      num_cores=_NC, num_subcores=_NS)
  return pl.kernel(
      _table_body,
      out_type=jax.ShapeDtypeStruct((_V, 2 * _C), jnp.float32),
      mesh=mesh,
      compiler_params=pltpu.CompilerParams(use_tc_tiling_on_sc=False),
      scratch_types=[
          pltpu.VMEM((2, _C, _TB + _L), jnp.float32),
          pltpu.VMEM((2, _TB // 2 + 8, 2 * _C), jnp.float32),
          pltpu.VMEM((2, _TB // 2, 2 * _C), jnp.float32),
          pltpu.SemaphoreType.DMA,
          pltpu.SemaphoreType.DMA,
          pltpu.SemaphoreType.DMA,
          pltpu.SemaphoreType.DMA,
      ],
  )


@functools.lru_cache(maxsize=None)
def _build(n_points):
  assert n_points % (_NW * _B * 2) == 0
  mesh = plsc.VectorSubcoreMesh(
      core_axis_name="c", subcore_axis_name="s",
      num_cores=_NC, num_subcores=_NS)
  return pl.kernel(
      _interp_body,
      out_type=jax.ShapeDtypeStruct((n_points * _C,), jnp.float32),
      mesh=mesh,
      compiler_params=pltpu.CompilerParams(use_tc_tiling_on_sc=False),
      scratch_types=[
          pltpu.VMEM((2, 3, _B), jnp.float32),
          pltpu.VMEM((2, 3, _B), jnp.float32),
          pltpu.VMEM((2, 4, _B), jnp.int32),
          pltpu.VMEM((2, 4, _B, 2 * _C), jnp.float32),
          pltpu.VMEM((2, _B * _C), jnp.float32),
          pltpu.SemaphoreType.DMA,
          pltpu.SemaphoreType.DMA,
          pltpu.SemaphoreType.DMA,
          pltpu.SemaphoreType.DMA,
          pltpu.SemaphoreType.DMA,
          pltpu.SemaphoreType.DMA,
      ],
  )


def kernel(x, grid, bound):
  n = x.shape[0]
  half = 0.5 * (_R - 1)
  xsc = (x.astype(jnp.float32) * (half / bound) + half).T
  table = _build_table()(grid.reshape(_C, _V))
  out = _build(n)(xsc[0], xsc[1], xsc[2], table)
  return out.reshape(n // _B, _C, _B).swapaxes(1, 2).reshape(n, _C)


# speedup vs baseline: 1.2258x; 1.0005x over previous
import functools

import jax
import jax.numpy as jnp
from jax import lax
from jax.experimental import pallas as pl
from jax.experimental.pallas import tpu as pltpu
from jax.experimental.pallas import tpu_sc as plsc

_C = 8
_R = 128
_V = _R * _R * _R
_NC = 2
_NS = 16
_NW = _NC * _NS
_L = 16
_B = 128


def _interp_body(xs, ys, zs, table, out, cv, fv, idxv, rowsv, outv,
                 csem0, csem1, gsem0, gsem1, osem0, osem1):
  n_points = xs.shape[0]
  pts_per_w = n_points // _NW
  n_chunks = pts_per_w // _B
  n_pairs = n_chunks // 2
  wid = lax.axis_index("s") * _NC + lax.axis_index("c")
  w_base = wid * pts_per_w

  csems = (csem0, csem1)
  gsems = (gsem0, gsem1)
  osems = (osem0, osem1)

  def coord_copies(t, par):
    base = w_base + t * _B
    sem = csems[par]
    return [
        pltpu.make_async_copy(xs.at[pl.ds(base, _B)], cv.at[par, 0], sem),
        pltpu.make_async_copy(ys.at[pl.ds(base, _B)], cv.at[par, 1], sem),
        pltpu.make_async_copy(zs.at[pl.ds(base, _B)], cv.at[par, 2], sem),
    ]

  def gather_copies(par):
    sem = gsems[par]
    return [
        pltpu.make_async_copy(table.at[idxv.at[par, k]], rowsv.at[par, k], sem)
        for k in range(4)
    ]

  def out_copy(t, par):
    base = w_base + t * _B
    return pltpu.make_async_copy(
        outv.at[par], out.at[pl.ds(base * _C, _B * _C)], osems[par])

  def fire(copies):
    for c in copies:
      c.start()

  def drain(copies):
    for c in copies:
      c.wait()

  def index_phase(par):
    def index_body(jo, _):
      for ji in range(2):
        index_group(par, 2 * jo + ji)
      return 0

    lax.fori_loop(0, _B // _L // 2, index_body, 0)

  def index_group(par, j):
      sl = pl.ds(j * _L, _L)
      sx = cv[par, 0, sl]
      sy = cv[par, 1, sl]
      sz = cv[par, 2, sl]
      ix = jnp.minimum(sx.astype(jnp.int32), _R - 2)
      iy = jnp.minimum(sy.astype(jnp.int32), _R - 2)
      iz = jnp.minimum(sz.astype(jnp.int32), _R - 2)
      fv[par, 0, sl] = sx - ix.astype(jnp.float32)
      fv[par, 1, sl] = sy - iy.astype(jnp.float32)
      fv[par, 2, sl] = sz - iz.astype(jnp.float32)
      cell = (iz << 14) + (iy << 7) + ix
      prow = (cell >> 1) + ((cell & 1) << 20)
      idxv[par, 0, sl] = prow
      idxv[par, 1, sl] = prow + 64
      idxv[par, 2, sl] = prow + 8192
      idxv[par, 3, sl] = prow + 8256

  lane = lax.iota(jnp.int32, _L)
  lo_half = lane < _C
  swap = lane ^ _C
  hi_f = (lane >> 3).astype(jnp.float32)

  def combine_phase(par):
    def combine_body(jj, _):
      sl = pl.ds(jj * _L, _L)
      fxr = fv[par, 0, sl]
      fyr = fv[par, 1, sl]
      fzr = fv[par, 2, sl]
      res = []
      for u in range(_L):
        splat = lane * 0 + u
        ex = fxr[splat]
        ey = fyr[splat]
        ez = fzr[splat]
        wx = jnp.where(lo_half, 1.0 - ex, ex)
        eyc = 1.0 - ey
        ezc = 1.0 - ez
        w00 = eyc * ezc
        w01 = ey * ezc
        w10 = eyc * ez
        w11 = ey * ez
        p = jj * _L + u
        tsum = (w00 * rowsv[par, 0, p, :] + w01 * rowsv[par, 1, p, :]
                + w10 * rowsv[par, 2, p, :] + w11 * rowsv[par, 3, p, :])
        acc = wx * tsum
        res.append(acc + acc[swap])
      m = [jnp.where(lo_half, res[q], res[q + 8]) for q in range(_C)]
      for b in range(3):
        nxt = list(m)
        rot = lane ^ (1 << b)
        bit = (lane >> b) & 1
        for u in range(_C):
          if u & (1 << b):
            continue
          v = u | (1 << b)
          xu, xv = m[u], m[v]
          nxt[u] = jnp.where(bit == 0, xu, xv[rot])
          nxt[v] = jnp.where(bit == 0, xu[rot], xv)
        m = nxt
      for c in range(_C):
        outv[par, pl.ds(c * _B + jj * _L, _L)] = m[c]
      return 0

    lax.fori_loop(0, _B // _L, combine_body, 0)

  fire(coord_copies(0, 0))
  fire(coord_copies(1, 1))

  def pair_body(m, _):
    a = 2 * m
    b = a + 1

    drain(coord_copies(a, 0))
    index_phase(0)
    fire(gather_copies(0))

    @pl.when(m + 1 < n_pairs)
    def _():
      fire(coord_copies(a + 2, 0))

    @pl.when(m > 0)
    def _():
      drain(gather_copies(1))

      @pl.when(m > 1)
      def _():
        drain([out_copy(b - 4, 1)])

      combine_phase(1)
      fire([out_copy(b - 2, 1)])

    drain(coord_copies(b, 1))
    index_phase(1)
    fire(gather_copies(1))

    @pl.when(m + 1 < n_pairs)
    def _():
      fire(coord_copies(b + 2, 1))

    drain(gather_copies(0))

    @pl.when(m > 0)
    def _():
      drain([out_copy(a - 2, 0)])

    combine_phase(0)
    fire([out_copy(a, 0)])
    return 0

  lax.fori_loop(0, n_pairs, pair_body, 0)

  last = n_chunks - 1
  drain(gather_copies(1))
  drain([out_copy(last - 2, 1)])
  combine_phase(1)
  fire([out_copy(last, 1)])
  drain([out_copy(last - 1, 0)])
  drain([out_copy(last, 1)])


_TB = 2048


def _table_body(g8v, table, inv, outve, outvo, isem0, isem1, osem0, osem1):
  nv = g8v.shape[1]
  cells_pw = nv // _NW
  n_ch = cells_pw // _TB
  nrow = _TB // 2
  wid = lax.axis_index("s") * _NC + lax.axis_index("c")
  cw = wid * cells_pw
  isems = (isem0, isem1)
  osems = (osem0, osem1)
  lane = lax.iota(jnp.int32, _L)
  lo_half = lane < _C
  swap = lane ^ _C

  def in_copies(t, par):
    c0 = cw + t * _TB
    ce = jnp.minimum(c0 + _TB, nv - _L)
    cps = [
        pltpu.make_async_copy(
            g8v.at[c, pl.ds(c0, _TB)], inv.at[par, c, pl.ds(0, _TB)],
            isems[par])
        for c in range(_C)
    ]
    cps += [
        pltpu.make_async_copy(
            g8v.at[c, pl.ds(ce, _L)], inv.at[par, c, pl.ds(_TB, _L)],
            isems[par])
        for c in range(_C)
    ]
    return cps

  def out_copies(t, par):
    r0 = (cw + t * _TB) // 2
    return [
        pltpu.make_async_copy(
            outve.at[par, pl.ds(0, nrow)], table.at[pl.ds(r0, nrow), :],
            osems[par]),
        pltpu.make_async_copy(
            outvo.at[par], table.at[pl.ds(nv // 2 + r0, nrow), :],
            osems[par]),
    ]

  def compute(par):
    def body(j, _):
      sl = pl.ds(j * _L, _L)
      m = [inv[par, c, sl] for c in range(_C)]
      for b in range(3):
        nxt = list(m)
        rot = lane ^ (1 << b)
        bit = (lane >> b) & 1
        for u in range(_C):
          if u & (1 << b):
            continue
          v = u | (1 << b)
          xu, xv = m[u], m[v]
          nxt[u] = jnp.where(bit == 0, xu, xv[rot])
          nxt[v] = jnp.where(bit == 0, xu[rot], xv)
        m = nxt
      nxt = list(m)
      bit = lane >> 3
      for u in range(0, _C, 2):
        xu, xv = m[u], m[u + 1]
        nxt[u] = jnp.where(bit == 0, xu, xv[swap])
        nxt[u + 1] = jnp.where(bit == 0, xu[swap], xv)
      m = nxt
      for q in range(_C):
        jrow = ((q & 1) << 2) | ((q >> 2) << 1) | ((q >> 1) & 1)
        outve[par, 8 * j + jrow, :] = m[q]
      return 0

    lax.fori_loop(0, _TB // _L + 1, body, 0)

    def odd_body(r, _):
      base = r * 8
      sw = [outve[par, base + i, :][swap] for i in range(9)]
      for i in range(8):
        outvo[par, base + i, :] = jnp.where(lo_half, sw[i], sw[i + 1])
      return 0

    lax.fori_loop(0, nrow // 8, odd_body, 0)

  fire = lambda cps: [c.start() for c in cps]
  drain = lambda cps: [c.wait() for c in cps]
  fire(in_copies(0, 0))
  fire(in_copies(1, 1))

  def pair_body(mm, _):
    a = 2 * mm
    b = a + 1
    drain(in_copies(a, 0))

    @pl.when(mm > 0)
    def _():
      drain(out_copies(a - 2, 0))

    compute(0)

    @pl.when(mm + 1 < n_ch // 2)
    def _():
      fire(in_copies(a + 2, 0))

    fire(out_copies(a, 0))
    drain(in_copies(b, 1))

    @pl.when(mm > 0)
    def _():
      drain(out_copies(b - 2, 1))

    compute(1)

    @pl.when(mm + 1 < n_ch // 2)
    def _():
      fire(in_copies(b + 2, 1))

    fire(out_copies(b, 1))
    return 0

  lax.fori_loop(0, n_ch // 2, pair_body, 0)
  drain(out_copies(n_ch - 2, 0))
  drain(out_copies(n_ch - 1, 1))


@functools.lru_cache(maxsize=None)
def _build_table():
  mesh = plsc.VectorSubcoreMesh(
      core_axis_name="c", subcore_axis_name="s",
      num_cores=_NC, num_subcores=_NS)
  return pl.kernel(
      _table_body,
      out_type=jax.ShapeDtypeStruct((_V, 2 * _C), jnp.float32),
      mesh=mesh,
      compiler_params=pltpu.CompilerParams(use_tc_tiling_on_sc=False),
      scratch_types=[
          pltpu.VMEM((2, _C, _TB + _L), jnp.float32),
          pltpu.VMEM((2, _TB // 2 + 8, 2 * _C), jnp.float32),
          pltpu.VMEM((2, _TB // 2, 2 * _C), jnp.float32),
          pltpu.SemaphoreType.DMA,
          pltpu.SemaphoreType.DMA,
          pltpu.SemaphoreType.DMA,
          pltpu.SemaphoreType.DMA,
      ],
  )


@functools.lru_cache(maxsize=None)
def _build(n_points):
  assert n_points % (_NW * _B * 2) == 0
  mesh = plsc.VectorSubcoreMesh(
      core_axis_name="c", subcore_axis_name="s",
      num_cores=_NC, num_subcores=_NS)
  return pl.kernel(
      _interp_body,
      out_type=jax.ShapeDtypeStruct((n_points * _C,), jnp.float32),
      mesh=mesh,
      compiler_params=pltpu.CompilerParams(use_tc_tiling_on_sc=False),
      scratch_types=[
          pltpu.VMEM((2, 3, _B), jnp.float32),
          pltpu.VMEM((2, 3, _B), jnp.float32),
          pltpu.VMEM((2, 4, _B), jnp.int32),
          pltpu.VMEM((2, 4, _B, 2 * _C), jnp.float32),
          pltpu.VMEM((2, _B * _C), jnp.float32),
          pltpu.SemaphoreType.DMA,
          pltpu.SemaphoreType.DMA,
          pltpu.SemaphoreType.DMA,
          pltpu.SemaphoreType.DMA,
          pltpu.SemaphoreType.DMA,
          pltpu.SemaphoreType.DMA,
      ],
  )


def kernel(x, grid, bound):
  n = x.shape[0]
  half = 0.5 * (_R - 1)
  xsc = (x.astype(jnp.float32) * (half / bound) + half).T
  table = _build_table()(grid.reshape(_C, _V))
  out = _build(n)(xsc[0], xsc[1], xsc[2], table)
  return out.reshape(n // _B, _C, _B).swapaxes(1, 2).reshape(n, _C)
